# baseline (speedup 1.0000x reference)
import jax
import jax.numpy as jnp
from jax import lax
from jax.experimental import pallas as pl
from jax.experimental.pallas import tpu as pltpu
from jax.experimental.pallas import tpu_sc as plsc

_NC = 2
_NS = 16
_NW = _NC * _NS
_L = 16

_BATCH = 16384
_D = 32
_BPW = _BATCH // _NW
_GROUPS = _BPW // _L


def _sc_body(users_hbm, items_hbm, uf_hbm, if_hbm, out_hbm,
             uidx_v, iidx_v, urows_v, irows_v, out_v, sem_u, sem_i):
    wid = lax.axis_index("s") * _NC + lax.axis_index("c")
    base = wid * _BPW

    pltpu.sync_copy(users_hbm.at[pl.ds(base, _BPW)], uidx_v)
    cp_u = pltpu.async_copy(uf_hbm.at[uidx_v], urows_v, sem_u)
    pltpu.sync_copy(items_hbm.at[pl.ds(base, _BPW)], iidx_v)
    cp_i = pltpu.async_copy(if_hbm.at[iidx_v], irows_v, sem_i)
    cp_u.wait()
    cp_i.wait()

    lane = lax.iota(jnp.int32, _L)

    def group(g, carry):
        row = g * _L + lane
        acc = jnp.zeros((_L,), jnp.float32)
        for k in range(_D):
            col = jnp.full((_L,), k, jnp.int32)
            gu = plsc.load_gather(urows_v, [row, col])
            gv = plsc.load_gather(irows_v, [row, col])
            acc = acc + gu * gv
        out_v[pl.ds(g * _L, _L)] = acc
        return carry

    lax.fori_loop(0, _GROUPS, group, 0)
    pltpu.sync_copy(out_v, out_hbm.at[pl.ds(base, _BPW)])


@jax.jit
def kernel(data, user_factors, item_factors):
    users = data[:, 0].astype(jnp.int32)
    items = data[:, 1].astype(jnp.int32)

    mesh = plsc.VectorSubcoreMesh(
        core_axis_name="c", subcore_axis_name="s",
        num_cores=_NC, num_subcores=_NS)
    run = pl.kernel(
        _sc_body,
        out_type=jax.ShapeDtypeStruct((_BATCH,), jnp.float32),
        mesh=mesh,
        scratch_types=[
            pltpu.VMEM((_BPW,), jnp.int32),
            pltpu.VMEM((_BPW,), jnp.int32),
            pltpu.VMEM((_BPW, _D), jnp.float32),
            pltpu.VMEM((_BPW, _D), jnp.float32),
            pltpu.VMEM((_BPW,), jnp.float32),
            pltpu.SemaphoreType.DMA,
            pltpu.SemaphoreType.DMA,
        ],
        compiler_params=pltpu.CompilerParams(
            needs_layout_passes=False, use_tc_tiling_on_sc=False),
    )
    return run(users, items, user_factors, item_factors)
